# Initial kernel scaffold; baseline (speedup 1.0000x reference)
import jax
import jax.numpy as jnp
from jax.experimental import pallas as pl


def kernel(x, table):
    raise NotImplementedError("write your pallas kernel here")


# speedup vs baseline: 6.3300x; 6.3300x over previous
import functools

import jax
import jax.numpy as jnp
from jax import lax
from jax.experimental import pallas as pl
from jax.experimental.pallas import tpu as pltpu
from jax.experimental.pallas import tpu_sc as plsc

D_MODEL = 32


@functools.partial(jax.jit, static_argnames=("b", "chunk"))
def _gather_sc(idx_flat, table, b, chunk):
    info = plsc.get_sparse_core_info()
    nw = info.num_cores * info.num_subcores
    b_per_w = b // nw
    n_chunks = b_per_w // chunk
    mesh = plsc.VectorSubcoreMesh(core_axis_name="c", subcore_axis_name="s")

    @functools.partial(
        pl.kernel,
        mesh=mesh,
        out_type=jax.ShapeDtypeStruct((b, D_MODEL), jnp.float32),
        scratch_types=[
            pltpu.VMEM((chunk,), jnp.int32),
            pltpu.VMEM((chunk, D_MODEL), jnp.float32),
            pltpu.SemaphoreType.DMA,
        ],
        compiler_params=pltpu.CompilerParams(use_tc_tiling_on_sc=False),
    )
    def k(idx_hbm, table_hbm, out_hbm, idx_v, rows_v, sem):
        wid = lax.axis_index("s") * info.num_cores + lax.axis_index("c")
        base = wid * b_per_w

        def body(g, carry):
            start = base + g * chunk
            pltpu.sync_copy(idx_hbm.at[pl.ds(start, chunk)], idx_v)
            pltpu.async_copy(table_hbm.at[idx_v], rows_v, sem).wait()
            pltpu.sync_copy(rows_v, out_hbm.at[pl.ds(start, chunk)])
            return carry

        lax.fori_loop(0, n_chunks, body, 0)

    return k(idx_flat, table)


def kernel(x, table):
    s0, s1 = x.shape
    b = s0 * s1
    idx_flat = x.reshape(b).astype(jnp.int32)
    out = _gather_sc(idx_flat, table, b, 2048)
    return out.reshape(s0, s1, D_MODEL)


# speedup vs baseline: 6.4924x; 1.0256x over previous
import functools

import jax
import jax.numpy as jnp
from jax import lax
from jax.experimental import pallas as pl
from jax.experimental.pallas import tpu as pltpu
from jax.experimental.pallas import tpu_sc as plsc

D_MODEL = 32


@functools.partial(jax.jit, static_argnames=("b", "chunk"))
def _gather_sc(idx_flat, table, b, chunk):
    info = plsc.get_sparse_core_info()
    nw = info.num_cores * info.num_subcores
    b_per_w = b // nw
    n_chunks = b_per_w // chunk
    assert n_chunks % 2 == 0
    mesh = plsc.VectorSubcoreMesh(core_axis_name="c", subcore_axis_name="s")

    @functools.partial(
        pl.kernel,
        mesh=mesh,
        out_type=jax.ShapeDtypeStruct((b, D_MODEL), jnp.float32),
        scratch_types=[
            pltpu.VMEM((chunk,), jnp.int32),
            pltpu.VMEM((chunk,), jnp.int32),
            pltpu.VMEM((chunk, D_MODEL), jnp.float32),
            pltpu.VMEM((chunk, D_MODEL), jnp.float32),
            pltpu.SemaphoreType.DMA,
            pltpu.SemaphoreType.DMA,
            pltpu.SemaphoreType.DMA,
            pltpu.SemaphoreType.DMA,
            pltpu.SemaphoreType.DMA,
            pltpu.SemaphoreType.DMA,
        ],
        compiler_params=pltpu.CompilerParams(use_tc_tiling_on_sc=False),
    )
    def k(idx_hbm, table_hbm, out_hbm, idx0, idx1, rows0, rows1,
          isem0, isem1, gsem0, gsem1, osem0, osem1):
        wid = lax.axis_index("s") * info.num_cores + lax.axis_index("c")
        base = wid * b_per_w
        idx_v = (idx0, idx1)
        rows_v = (rows0, rows1)
        isem = (isem0, isem1)
        gsem = (gsem0, gsem1)
        osem = (osem0, osem1)

        pltpu.async_copy(idx_hbm.at[pl.ds(base, chunk)], idx0, isem0)
        pltpu.async_copy(idx_hbm.at[pl.ds(base + chunk, chunk)], idx1, isem1)

        def body(h, carry):
            for bi in range(2):
                g = 2 * h + bi
                start = base + g * chunk
                pltpu.make_async_copy(
                    idx_hbm.at[pl.ds(start, chunk)], idx_v[bi], isem[bi]).wait()

                @pl.when(g >= 2)
                def _():
                    pltpu.make_async_copy(
                        rows_v[bi],
                        out_hbm.at[pl.ds(start - 2 * chunk, chunk)],
                        osem[bi]).wait()

                gather = pltpu.async_copy(
                    table_hbm.at[idx_v[bi]], rows_v[bi], gsem[bi])
                gather.wait()

                @pl.when(g + 2 < n_chunks)
                def _():
                    pltpu.async_copy(
                        idx_hbm.at[pl.ds(start + 2 * chunk, chunk)],
                        idx_v[bi], isem[bi])

                pltpu.async_copy(
                    rows_v[bi], out_hbm.at[pl.ds(start, chunk)], osem[bi])
            return carry

        lax.fori_loop(0, n_chunks // 2, body, 0)

        last = base + (n_chunks - 2) * chunk
        pltpu.make_async_copy(
            rows0, out_hbm.at[pl.ds(last, chunk)], osem0).wait()
        pltpu.make_async_copy(
            rows1, out_hbm.at[pl.ds(last + chunk, chunk)], osem1).wait()

    return k(idx_flat, table)


def kernel(x, table):
    s0, s1 = x.shape
    b = s0 * s1
    idx_flat = x.reshape(b).astype(jnp.int32)
    out = _gather_sc(idx_flat, table, b, 1600)
    return out.reshape(s0, s1, D_MODEL)
